# baseline (speedup 1.0000x reference)
import functools

import jax
import jax.numpy as jnp
from jax.experimental import pallas as pl

_B, _LC, _LQ, _LCH = 8, 512, 64, 16
_NH = 16
_HID = 128

_INTERPRET = False


def _f32(x):
    return x.astype(jnp.float32)


def _ln_in(x, g, b):
    m = jnp.mean(x, -1, keepdims=True)
    v = jnp.var(x, -1, keepdims=True)
    return (x - m) / jnp.sqrt(v + 1e-5) * g + b


def _lsh_body(x_ref, mask_ref, maskc_ref, ln1g, ln1b, wqk, wv, wo, rot,
              ln2g, ln2b, ff1w, ff1b, ff2w, ff2b, out_ref, *, heads, L, D):
    x = x_ref[0]
    mask_row = mask_ref[0]
    mask_col = maskc_ref[0]
    h = _ln_in(x, ln1g[...], ln1b[...])
    qk_all = jnp.dot(h, wqk[...], preferred_element_type=jnp.float32)
    v_all = jnp.dot(h, wv[...], preferred_element_type=jnp.float32)
    dh = D // heads
    rot_all = rot[...]
    row = jax.lax.broadcasted_iota(jnp.int32, (L, L), 0)
    col = jax.lax.broadcasted_iota(jnp.int32, (L, L), 1)
    causal = (row >= col) & (mask_row > 0.0)
    diag = row == col
    ids32 = jax.lax.broadcasted_iota(jnp.int32, (L, 32), 1)
    scale = 1.0 / jnp.sqrt(float(dh))
    outs = []
    for hd in range(heads):
        qk = qk_all[:, hd * dh:(hd + 1) * dh]
        vh = v_all[:, hd * dh:(hd + 1) * dh]
        nrm = jnp.sqrt(jnp.sum(qk * qk, axis=1, keepdims=True))
        kn = qk / (nrm + 1e-6)
        proj = jnp.dot(qk, rot_all, preferred_element_type=jnp.float32)
        allowed = None
        for r in range(_NH):
            pr = proj[:, r * 32:(r + 1) * 32]
            mxp = jnp.max(pr, axis=1, keepdims=True)
            mxn = jnp.max(-pr, axis=1, keepdims=True)
            idp = jnp.min(jnp.where(pr >= mxp, ids32, 32),
                          axis=1, keepdims=True)
            idn = jnp.min(jnp.where(-pr >= mxn, ids32, 32),
                          axis=1, keepdims=True)
            bk = jnp.where(mxp >= mxn, idp, idn + 32)
            bkf = bk.astype(jnp.float32)
            bkT = jax.lax.transpose(bkf, (1, 0))
            eq = bkf == bkT
            allowed = eq if allowed is None else (allowed | eq)
        allowed = allowed & causal
        scores = jax.lax.dot_general(
            qk, kn, (((1,), (1,)), ((), ())),
            preferred_element_type=jnp.float32) * scale
        scores = jnp.where(allowed, scores, -1e9)
        scores = jnp.where(diag, -1e5, scores)
        mxs = jnp.max(scores, axis=1, keepdims=True)
        e = jnp.exp(scores - mxs)
        attn = e / jnp.sum(e, axis=1, keepdims=True)
        outs.append(jnp.dot(attn, vh, preferred_element_type=jnp.float32))
    o = jnp.concatenate(outs, axis=1)
    o = jnp.dot(o, wo[...], preferred_element_type=jnp.float32)
    x = x + o * mask_col
    h2 = _ln_in(x, ln2g[...], ln2b[...])
    ff = jax.nn.gelu(jnp.dot(h2, ff1w[...],
                             preferred_element_type=jnp.float32) + ff1b[...])
    ff = jnp.dot(ff, ff2w[...], preferred_element_type=jnp.float32) + ff2b[...]
    out_ref[0] = x + ff


def _lsh_block(x, mask, p, heads):
    Bb, L, D = x.shape
    dh = D // heads
    rot = jnp.transpose(p['rot'], (1, 0, 2)).reshape(dh, _NH * 32)
    mask3 = mask.reshape(Bb, 1, L)
    maskc3 = mask.reshape(Bb, L, 1)
    row1 = lambda a: a.reshape(1, -1)
    args = (x, mask3, maskc3,
            row1(p['ln1_g']), row1(p['ln1_b']),
            p['wqk'], p['wv'], p['wo'], rot,
            row1(p['ln2_g']), row1(p['ln2_b']),
            p['ff1_w'], row1(p['ff1_b']), p['ff2_w'], row1(p['ff2_b']))
    batch_spec_x = pl.BlockSpec((1, L, D), lambda b: (b, 0, 0))
    batch_spec_m = pl.BlockSpec((1, 1, L), lambda b: (b, 0, 0))
    batch_spec_mc = pl.BlockSpec((1, L, 1), lambda b: (b, 0, 0))
    w_spec = lambda a: pl.BlockSpec(a.shape, lambda b, _n=a.ndim: (0,) * _n)
    in_specs = ([batch_spec_x, batch_spec_m, batch_spec_mc]
                + [w_spec(a) for a in args[3:]])
    return pl.pallas_call(
        functools.partial(_lsh_body, heads=heads, L=L, D=D),
        grid=(Bb,),
        in_specs=in_specs,
        out_specs=batch_spec_x,
        out_shape=jax.ShapeDtypeStruct((Bb, L, D), jnp.float32),
        interpret=_INTERPRET,
    )(*args)


def _embed_body(cc_ref, cw_ref, char_w, char_b, pw_a, pw_b, proj_b,
                hw0, hb0, gw0, gb0, hw1, hb1, gw1, gb1, out_ref, *, L):
    cw = cw_ref[0]
    ch = None
    for k in range(_LCH):
        cck = cc_ref[0, :, k, :]
        t = jax.nn.relu(jnp.dot(cck, char_w[...],
                                preferred_element_type=jnp.float32)
                        + char_b[...])
        ch = t if ch is None else jnp.maximum(ch, t)
    x = (jnp.dot(cw, pw_a[...], preferred_element_type=jnp.float32)
         + jnp.dot(ch, pw_b[...], preferred_element_type=jnp.float32)
         + proj_b[...])
    for hw, hb, gw, gb in ((hw0, hb0, gw0, gb0), (hw1, hb1, gw1, gb1)):
        g = jax.nn.sigmoid(jnp.dot(x, gw[...],
                                   preferred_element_type=jnp.float32)
                           + gb[...])
        h = jax.nn.relu(jnp.dot(x, hw[...],
                                preferred_element_type=jnp.float32)
                        + hb[...])
        x = g * h + (1.0 - g) * x
    out_ref[0] = x


def _embed(cc, cw, p):
    Bb, L = cw.shape[0], cw.shape[1]
    DW = cw.shape[2]
    DC = cc.shape[3]
    row1 = lambda a: a.reshape(1, -1)
    pw = p['proj_w']
    args = (cc, cw, p['char_w'], row1(p['char_b']),
            pw[:DW], pw[DW:], row1(p['proj_b']),
            p['hw0'], row1(p['hb0']), p['gw0'], row1(p['gb0']),
            p['hw1'], row1(p['hb1']), p['gw1'], row1(p['gb1']))
    cc_spec = pl.BlockSpec((1, L, _LCH, DC), lambda b: (b, 0, 0, 0))
    cw_spec = pl.BlockSpec((1, L, DW), lambda b: (b, 0, 0))
    w_spec = lambda a: pl.BlockSpec(a.shape, lambda b, _n=a.ndim: (0,) * _n)
    in_specs = [cc_spec, cw_spec] + [w_spec(a) for a in args[2:]]
    return pl.pallas_call(
        functools.partial(_embed_body, L=L),
        grid=(Bb,),
        in_specs=in_specs,
        out_specs=pl.BlockSpec((1, L, _HID), lambda b: (b, 0, 0)),
        out_shape=jax.ShapeDtypeStruct((Bb, L, _HID), jnp.float32),
        interpret=_INTERPRET,
    )(*args)


def _bidaf_body(ce_ref, qe_ref, mc_ref, mq_ref, w4c, w4q, w4m, cqb,
                rzw, rzb, out_ref):
    ce = ce_ref[0]
    qe = qe_ref[0]
    mc_col = mc_ref[0]
    mq = mq_ref[0]
    sc = jnp.dot(ce, w4c[...], preferred_element_type=jnp.float32)
    sq = jax.lax.dot_general(
        w4q[...], qe, (((0,), (1,)), ((), ())),
        preferred_element_type=jnp.float32)
    sm = jax.lax.dot_general(
        ce * jnp.reshape(w4m[...], (1, -1)), qe,
        (((1,), (1,)), ((), ())), preferred_element_type=jnp.float32)
    S = sc + sq + sm + cqb[0, 0]
    s1 = jnp.where(mq > 0.0, S, -1e30)
    m1 = jnp.max(s1, axis=1, keepdims=True)
    e1 = jnp.exp(s1 - m1)
    S1 = e1 / jnp.sum(e1, axis=1, keepdims=True)
    s2 = jnp.where(mc_col > 0.0, S, -1e30)
    m2 = jnp.max(s2, axis=0, keepdims=True)
    e2 = jnp.exp(s2 - m2)
    S2 = e2 / jnp.sum(e2, axis=0, keepdims=True)
    A = jnp.dot(S1, qe, preferred_element_type=jnp.float32)
    t = jax.lax.dot_general(
        S2, ce, (((0,), (0,)), ((), ())), preferred_element_type=jnp.float32)
    Bm = jnp.dot(S1, t, preferred_element_type=jnp.float32)
    D = ce.shape[1]
    rz = rzw[...]
    r0 = rz[0 * D:1 * D]
    r1 = rz[1 * D:2 * D]
    r2 = rz[2 * D:3 * D]
    r3 = rz[3 * D:4 * D]
    M0 = (jnp.dot(ce, r0, preferred_element_type=jnp.float32)
          + jnp.dot(A, r1, preferred_element_type=jnp.float32)
          + jnp.dot(ce * A, r2, preferred_element_type=jnp.float32)
          + jnp.dot(ce * Bm, r3, preferred_element_type=jnp.float32)
          + rzb[...])
    out_ref[0] = M0


def _bidaf(ce, qe, mask_c, mask_q, cq, rz_w, rz_b):
    Bb, LC, D = ce.shape
    LQ = qe.shape[1]
    args = (ce, qe, mask_c.reshape(Bb, LC, 1), mask_q.reshape(Bb, 1, LQ),
            cq['w4c'].reshape(-1, 1), cq['w4q'].reshape(-1, 1),
            cq['w4m'].reshape(1, -1), cq['cq_b'].reshape(1, 1),
            rz_w, rz_b.reshape(1, -1))
    specs = [pl.BlockSpec((1, LC, D), lambda b: (b, 0, 0)),
             pl.BlockSpec((1, LQ, D), lambda b: (b, 0, 0)),
             pl.BlockSpec((1, LC, 1), lambda b: (b, 0, 0)),
             pl.BlockSpec((1, 1, LQ), lambda b: (b, 0, 0))]
    w_spec = lambda a: pl.BlockSpec(a.shape, lambda b, _n=a.ndim: (0,) * _n)
    specs += [w_spec(a) for a in args[4:]]
    return pl.pallas_call(
        _bidaf_body,
        grid=(Bb,),
        in_specs=specs,
        out_specs=pl.BlockSpec((1, LC, D), lambda b: (b, 0, 0)),
        out_shape=jax.ShapeDtypeStruct((Bb, LC, D), jnp.float32),
        interpret=_INTERPRET,
    )(*args)


def _logits_body(m1_ref, m2_ref, m3_ref, mc_ref, p1a, p1b, p2a, p2b,
                 out1_ref, out2_ref):
    m1 = m1_ref[0]
    m2 = m2_ref[0]
    m3 = m3_ref[0]
    mc = mc_ref[0]
    def head(ma, mb, wa, wb):
        la = jax.lax.dot_general(wa[...], ma, (((0,), (1,)), ((), ())),
                                 preferred_element_type=jnp.float32)
        lb = jax.lax.dot_general(wb[...], mb, (((0,), (1,)), ((), ())),
                                 preferred_element_type=jnp.float32)
        l = jnp.where(mc > 0.0, la + lb, -1e30)
        mx = jnp.max(l, axis=1, keepdims=True)
        return l - mx - jnp.log(jnp.sum(jnp.exp(l - mx), axis=1,
                                        keepdims=True))
    out1_ref[0] = head(m1, m2, p1a, p1b)
    out2_ref[0] = head(m1, m3, p2a, p2b)


def _logits(m1, m2, m3, mask_c, p1_w, p2_w):
    Bb, LC, D = m1.shape
    args = (m1, m2, m3, mask_c.reshape(Bb, 1, LC),
            p1_w[:D].reshape(-1, 1), p1_w[D:].reshape(-1, 1),
            p2_w[:D].reshape(-1, 1), p2_w[D:].reshape(-1, 1))
    x_spec = pl.BlockSpec((1, LC, D), lambda b: (b, 0, 0))
    m_spec = pl.BlockSpec((1, 1, LC), lambda b: (b, 0, 0))
    w_spec = lambda a: pl.BlockSpec(a.shape, lambda b, _n=a.ndim: (0,) * _n)
    specs = [x_spec, x_spec, x_spec, m_spec] + [w_spec(a) for a in args[4:]]
    o_spec = pl.BlockSpec((1, 1, LC), lambda b: (b, 0, 0))
    o1, o2 = pl.pallas_call(
        _logits_body,
        grid=(Bb,),
        in_specs=specs,
        out_specs=[o_spec, o_spec],
        out_shape=[jax.ShapeDtypeStruct((Bb, 1, LC), jnp.float32),
                   jax.ShapeDtypeStruct((Bb, 1, LC), jnp.float32)],
        interpret=_INTERPRET,
    )(*args)
    return o1.reshape(Bb, LC), o2.reshape(Bb, LC)


def kernel(Cwid, Qwid, Ccid, Qcid, CQid, params):
    p = params
    maskC = (Cwid != 0).astype(jnp.float32)
    maskQ = (Qwid != 0).astype(jnp.float32)
    Cw = jnp.take(p['word_table'], Cwid, axis=0)
    Qw = jnp.take(p['word_table'], Qwid, axis=0)
    Cc = jnp.take(p['char_table'], Ccid, axis=0)
    Qc = jnp.take(p['char_table'], Qcid, axis=0)
    C = _embed(Cc, Cw, p['emb'])
    Q = _embed(Qc, Qw, p['emb'])
    Ce = _lsh_block(C, maskC, p['enc'], 4)
    Qe = _lsh_block(Q, maskQ, p['enc'], 4)
    M0 = _bidaf(Ce, Qe, maskC, maskQ, p['cq'], p['rz_w'], p['rz_b'])
    for blk in p['blocks']:
        M0 = _lsh_block(M0, maskC, blk, 8)
    M1 = M0
    for blk in p['blocks']:
        M0 = _lsh_block(M0, maskC, blk, 8)
    M2 = M0
    for blk in p['blocks']:
        M0 = _lsh_block(M0, maskC, blk, 8)
    M3 = M0
    return _logits(M1, M2, M3, maskC, p['out']['p1_w'], p['out']['p2_w'])


# speedup vs baseline: 1.1754x; 1.1754x over previous
import functools

import jax
import jax.numpy as jnp
from jax.experimental import pallas as pl

_B, _LC, _LQ, _LCH = 8, 512, 64, 16
_NH = 16
_HID = 128

_INTERPRET = False


def _f32(x):
    return x.astype(jnp.float32)


def _ln_in(x, g, b):
    m = jnp.mean(x, -1, keepdims=True)
    v = jnp.var(x, -1, keepdims=True)
    return (x - m) / jnp.sqrt(v + 1e-5) * g + b


def _lsh_body(x_ref, mask_ref, maskc_ref, ln1g, ln1b, wqk, wv, wo, wrot,
              ln2g, ln2b, ff1w, ff1b, ff2w, ff2b, out_ref, *, heads, L, D):
    x = x_ref[0]
    mask_row = mask_ref[0]
    mask_col = maskc_ref[0]
    h = _ln_in(x, ln1g[...], ln1b[...])
    qk_all = jnp.dot(h, wqk[...], preferred_element_type=jnp.float32)
    v_all = jnp.dot(h, wv[...], preferred_element_type=jnp.float32)
    dh = D // heads
    NR = _NH * 32
    lr = jax.lax.broadcasted_iota(jnp.int32, (D, D), 0) // dh
    lc = jax.lax.broadcasted_iota(jnp.int32, (D, D), 1) // dh
    band_ones = (lr == lc).astype(jnp.float32)
    norm2 = jnp.dot(qk_all * qk_all, band_ones,
                    preferred_element_type=jnp.float32)
    kn_all = qk_all / (jnp.sqrt(norm2) + 1e-6)
    proj = jnp.dot(qk_all, wrot[...],
                   preferred_element_type=jnp.float32)
    projT = jax.lax.transpose(proj, (1, 0))
    row = jax.lax.broadcasted_iota(jnp.int32, (L, L), 0)
    col = jax.lax.broadcasted_iota(jnp.int32, (L, L), 1)
    causal = (row >= col) & (mask_row > 0.0)
    diag = row == col
    ids32 = jax.lax.broadcasted_iota(jnp.int32, (L, 32), 1)
    ids32T = jax.lax.broadcasted_iota(jnp.int32, (32, L), 0)
    lane_head = jax.lax.broadcasted_iota(jnp.int32, (1, D), 1) // dh
    scale = 1.0 / jnp.sqrt(float(dh))
    knt_pieces = []
    v_pieces = []
    for hd in range(heads):
        hm = (lane_head == hd).astype(jnp.float32)
        knt_pieces.append(jax.lax.transpose(kn_all * hm, (1, 0)))
        v_pieces.append(v_all * hm)
    knbig = jnp.concatenate(knt_pieces, axis=1)
    vstack = jnp.concatenate(v_pieces, axis=0)
    scores_full = jnp.dot(qk_all, knbig,
                          preferred_element_type=jnp.float32)
    attn_pieces = []
    for hd in range(heads):
        allowed = None
        for r in range(_NH):
            off = hd * NR + r * 32
            pr = proj[:, off:off + 32]
            mxp = jnp.max(pr, axis=1, keepdims=True)
            mxn = jnp.max(-pr, axis=1, keepdims=True)
            idp = jnp.min(jnp.where(pr >= mxp, ids32, 32),
                          axis=1, keepdims=True)
            idn = jnp.min(jnp.where(-pr >= mxn, ids32, 32),
                          axis=1, keepdims=True)
            bk = jnp.where(mxp >= mxn, idp, idn + 32)
            prT = projT[off:off + 32, :]
            mxpT = jnp.max(prT, axis=0, keepdims=True)
            mxnT = jnp.max(-prT, axis=0, keepdims=True)
            idpT = jnp.min(jnp.where(prT >= mxpT, ids32T, 32),
                           axis=0, keepdims=True)
            idnT = jnp.min(jnp.where(-prT >= mxnT, ids32T, 32),
                           axis=0, keepdims=True)
            bkT = jnp.where(mxpT >= mxnT, idpT, idnT + 32)
            eq = bk == bkT
            allowed = eq if allowed is None else (allowed | eq)
        allowed = allowed & causal
        s = scores_full[:, hd * L:(hd + 1) * L] * scale
        s = jnp.where(allowed, s, -1e9)
        s = jnp.where(diag, -1e5, s)
        mxs = jnp.max(s, axis=1, keepdims=True)
        e = jnp.exp(s - mxs)
        attn_pieces.append(e / jnp.sum(e, axis=1, keepdims=True))
    attn_full = jnp.concatenate(attn_pieces, axis=1)
    o = jnp.dot(attn_full, vstack, preferred_element_type=jnp.float32)
    o = jnp.dot(o, wo[...], preferred_element_type=jnp.float32)
    x = x + o * mask_col
    h2 = _ln_in(x, ln2g[...], ln2b[...])
    ff = jax.nn.gelu(jnp.dot(h2, ff1w[...],
                             preferred_element_type=jnp.float32) + ff1b[...])
    ff = jnp.dot(ff, ff2w[...], preferred_element_type=jnp.float32) + ff2b[...]
    out_ref[0] = x + ff


def _lsh_block(x, mask, p, heads):
    Bb, L, D = x.shape
    dh = D // heads
    rot = jnp.transpose(p['rot'], (1, 0, 2)).reshape(dh, _NH * 32)
    rot = jnp.kron(jnp.eye(heads, dtype=jnp.float32), rot)
    mask3 = mask.reshape(Bb, 1, L)
    maskc3 = mask.reshape(Bb, L, 1)
    row1 = lambda a: a.reshape(1, -1)
    args = (x, mask3, maskc3,
            row1(p['ln1_g']), row1(p['ln1_b']),
            p['wqk'], p['wv'], p['wo'], rot,
            row1(p['ln2_g']), row1(p['ln2_b']),
            p['ff1_w'], row1(p['ff1_b']), p['ff2_w'], row1(p['ff2_b']))
    batch_spec_x = pl.BlockSpec((1, L, D), lambda b: (b, 0, 0))
    batch_spec_m = pl.BlockSpec((1, 1, L), lambda b: (b, 0, 0))
    batch_spec_mc = pl.BlockSpec((1, L, 1), lambda b: (b, 0, 0))
    w_spec = lambda a: pl.BlockSpec(a.shape, lambda b, _n=a.ndim: (0,) * _n)
    in_specs = ([batch_spec_x, batch_spec_m, batch_spec_mc]
                + [w_spec(a) for a in args[3:]])
    return pl.pallas_call(
        functools.partial(_lsh_body, heads=heads, L=L, D=D),
        grid=(Bb,),
        in_specs=in_specs,
        out_specs=batch_spec_x,
        out_shape=jax.ShapeDtypeStruct((Bb, L, D), jnp.float32),
        interpret=_INTERPRET,
    )(*args)


def _embed_body(cc_ref, cw_ref, char_w, char_b, pw_a, pw_b, proj_b,
                hw0, hb0, gw0, gb0, hw1, hb1, gw1, gb1, out_ref, *, L):
    cw = cw_ref[0]
    ch = None
    for k in range(_LCH):
        cck = cc_ref[0, :, k, :]
        t = jax.nn.relu(jnp.dot(cck, char_w[...],
                                preferred_element_type=jnp.float32)
                        + char_b[...])
        ch = t if ch is None else jnp.maximum(ch, t)
    x = (jnp.dot(cw, pw_a[...], preferred_element_type=jnp.float32)
         + jnp.dot(ch, pw_b[...], preferred_element_type=jnp.float32)
         + proj_b[...])
    for hw, hb, gw, gb in ((hw0, hb0, gw0, gb0), (hw1, hb1, gw1, gb1)):
        g = jax.nn.sigmoid(jnp.dot(x, gw[...],
                                   preferred_element_type=jnp.float32)
                           + gb[...])
        h = jax.nn.relu(jnp.dot(x, hw[...],
                                preferred_element_type=jnp.float32)
                        + hb[...])
        x = g * h + (1.0 - g) * x
    out_ref[0] = x


def _embed(cc, cw, p):
    Bb, L = cw.shape[0], cw.shape[1]
    DW = cw.shape[2]
    DC = cc.shape[3]
    row1 = lambda a: a.reshape(1, -1)
    pw = p['proj_w']
    args = (cc, cw, p['char_w'], row1(p['char_b']),
            pw[:DW], pw[DW:], row1(p['proj_b']),
            p['hw0'], row1(p['hb0']), p['gw0'], row1(p['gb0']),
            p['hw1'], row1(p['hb1']), p['gw1'], row1(p['gb1']))
    cc_spec = pl.BlockSpec((1, L, _LCH, DC), lambda b: (b, 0, 0, 0))
    cw_spec = pl.BlockSpec((1, L, DW), lambda b: (b, 0, 0))
    w_spec = lambda a: pl.BlockSpec(a.shape, lambda b, _n=a.ndim: (0,) * _n)
    in_specs = [cc_spec, cw_spec] + [w_spec(a) for a in args[2:]]
    return pl.pallas_call(
        functools.partial(_embed_body, L=L),
        grid=(Bb,),
        in_specs=in_specs,
        out_specs=pl.BlockSpec((1, L, _HID), lambda b: (b, 0, 0)),
        out_shape=jax.ShapeDtypeStruct((Bb, L, _HID), jnp.float32),
        interpret=_INTERPRET,
    )(*args)


def _bidaf_body(ce_ref, qe_ref, mc_ref, mq_ref, w4c, w4q, w4m, cqb,
                rzw, rzb, out_ref):
    ce = ce_ref[0]
    qe = qe_ref[0]
    mc_col = mc_ref[0]
    mq = mq_ref[0]
    sc = jnp.dot(ce, w4c[...], preferred_element_type=jnp.float32)
    sq = jax.lax.dot_general(
        w4q[...], qe, (((0,), (1,)), ((), ())),
        preferred_element_type=jnp.float32)
    sm = jax.lax.dot_general(
        ce * jnp.reshape(w4m[...], (1, -1)), qe,
        (((1,), (1,)), ((), ())), preferred_element_type=jnp.float32)
    S = sc + sq + sm + cqb[0, 0]
    s1 = jnp.where(mq > 0.0, S, -1e30)
    m1 = jnp.max(s1, axis=1, keepdims=True)
    e1 = jnp.exp(s1 - m1)
    S1 = e1 / jnp.sum(e1, axis=1, keepdims=True)
    s2 = jnp.where(mc_col > 0.0, S, -1e30)
    m2 = jnp.max(s2, axis=0, keepdims=True)
    e2 = jnp.exp(s2 - m2)
    S2 = e2 / jnp.sum(e2, axis=0, keepdims=True)
    A = jnp.dot(S1, qe, preferred_element_type=jnp.float32)
    t = jax.lax.dot_general(
        S2, ce, (((0,), (0,)), ((), ())), preferred_element_type=jnp.float32)
    Bm = jnp.dot(S1, t, preferred_element_type=jnp.float32)
    D = ce.shape[1]
    rz = rzw[...]
    r0 = rz[0 * D:1 * D]
    r1 = rz[1 * D:2 * D]
    r2 = rz[2 * D:3 * D]
    r3 = rz[3 * D:4 * D]
    M0 = (jnp.dot(ce, r0, preferred_element_type=jnp.float32)
          + jnp.dot(A, r1, preferred_element_type=jnp.float32)
          + jnp.dot(ce * A, r2, preferred_element_type=jnp.float32)
          + jnp.dot(ce * Bm, r3, preferred_element_type=jnp.float32)
          + rzb[...])
    out_ref[0] = M0


def _bidaf(ce, qe, mask_c, mask_q, cq, rz_w, rz_b):
    Bb, LC, D = ce.shape
    LQ = qe.shape[1]
    args = (ce, qe, mask_c.reshape(Bb, LC, 1), mask_q.reshape(Bb, 1, LQ),
            cq['w4c'].reshape(-1, 1), cq['w4q'].reshape(-1, 1),
            cq['w4m'].reshape(1, -1), cq['cq_b'].reshape(1, 1),
            rz_w, rz_b.reshape(1, -1))
    specs = [pl.BlockSpec((1, LC, D), lambda b: (b, 0, 0)),
             pl.BlockSpec((1, LQ, D), lambda b: (b, 0, 0)),
             pl.BlockSpec((1, LC, 1), lambda b: (b, 0, 0)),
             pl.BlockSpec((1, 1, LQ), lambda b: (b, 0, 0))]
    w_spec = lambda a: pl.BlockSpec(a.shape, lambda b, _n=a.ndim: (0,) * _n)
    specs += [w_spec(a) for a in args[4:]]
    return pl.pallas_call(
        _bidaf_body,
        grid=(Bb,),
        in_specs=specs,
        out_specs=pl.BlockSpec((1, LC, D), lambda b: (b, 0, 0)),
        out_shape=jax.ShapeDtypeStruct((Bb, LC, D), jnp.float32),
        interpret=_INTERPRET,
    )(*args)


def _logits_body(m1_ref, m2_ref, m3_ref, mc_ref, p1a, p1b, p2a, p2b,
                 out1_ref, out2_ref):
    m1 = m1_ref[0]
    m2 = m2_ref[0]
    m3 = m3_ref[0]
    mc = mc_ref[0]
    def head(ma, mb, wa, wb):
        la = jax.lax.dot_general(wa[...], ma, (((0,), (1,)), ((), ())),
                                 preferred_element_type=jnp.float32)
        lb = jax.lax.dot_general(wb[...], mb, (((0,), (1,)), ((), ())),
                                 preferred_element_type=jnp.float32)
        l = jnp.where(mc > 0.0, la + lb, -1e30)
        mx = jnp.max(l, axis=1, keepdims=True)
        return l - mx - jnp.log(jnp.sum(jnp.exp(l - mx), axis=1,
                                        keepdims=True))
    out1_ref[0] = head(m1, m2, p1a, p1b)
    out2_ref[0] = head(m1, m3, p2a, p2b)


def _logits(m1, m2, m3, mask_c, p1_w, p2_w):
    Bb, LC, D = m1.shape
    args = (m1, m2, m3, mask_c.reshape(Bb, 1, LC),
            p1_w[:D].reshape(-1, 1), p1_w[D:].reshape(-1, 1),
            p2_w[:D].reshape(-1, 1), p2_w[D:].reshape(-1, 1))
    x_spec = pl.BlockSpec((1, LC, D), lambda b: (b, 0, 0))
    m_spec = pl.BlockSpec((1, 1, LC), lambda b: (b, 0, 0))
    w_spec = lambda a: pl.BlockSpec(a.shape, lambda b, _n=a.ndim: (0,) * _n)
    specs = [x_spec, x_spec, x_spec, m_spec] + [w_spec(a) for a in args[4:]]
    o_spec = pl.BlockSpec((1, 1, LC), lambda b: (b, 0, 0))
    o1, o2 = pl.pallas_call(
        _logits_body,
        grid=(Bb,),
        in_specs=specs,
        out_specs=[o_spec, o_spec],
        out_shape=[jax.ShapeDtypeStruct((Bb, 1, LC), jnp.float32),
                   jax.ShapeDtypeStruct((Bb, 1, LC), jnp.float32)],
        interpret=_INTERPRET,
    )(*args)
    return o1.reshape(Bb, LC), o2.reshape(Bb, LC)


def kernel(Cwid, Qwid, Ccid, Qcid, CQid, params):
    p = params
    maskC = (Cwid != 0).astype(jnp.float32)
    maskQ = (Qwid != 0).astype(jnp.float32)
    Cw = jnp.take(p['word_table'], Cwid, axis=0)
    Qw = jnp.take(p['word_table'], Qwid, axis=0)
    Cc = jnp.take(p['char_table'], Ccid, axis=0)
    Qc = jnp.take(p['char_table'], Qcid, axis=0)
    C = _embed(Cc, Cw, p['emb'])
    Q = _embed(Qc, Qw, p['emb'])
    Ce = _lsh_block(C, maskC, p['enc'], 4)
    Qe = _lsh_block(Q, maskQ, p['enc'], 4)
    M0 = _bidaf(Ce, Qe, maskC, maskQ, p['cq'], p['rz_w'], p['rz_b'])
    for blk in p['blocks']:
        M0 = _lsh_block(M0, maskC, blk, 8)
    M1 = M0
    for blk in p['blocks']:
        M0 = _lsh_block(M0, maskC, blk, 8)
    M2 = M0
    for blk in p['blocks']:
        M0 = _lsh_block(M0, maskC, blk, 8)
    M3 = M0
    return _logits(M1, M2, M3, maskC, p['out']['p1_w'], p['out']['p2_w'])


# speedup vs baseline: 4.9290x; 4.1935x over previous
import functools

import jax
import jax.numpy as jnp
from jax.experimental import pallas as pl

_B, _LC, _LQ, _LCH = 8, 512, 64, 16
_NH = 16
_HID = 128

_INTERPRET = False


def _f32(x):
    return x.astype(jnp.float32)


def _ln_in(x, g, b):
    m = jnp.mean(x, -1, keepdims=True)
    v = jnp.var(x, -1, keepdims=True)
    return (x - m) / jnp.sqrt(v + 1e-5) * g + b


def _lsh_body(x_ref, mask_ref, maskc_ref, ln1g, ln1b, wqk, wv, wo, wrot,
              ln2g, ln2b, ff1w, ff1b, ff2w, ff2b, out_ref, *, heads, L, D):
    x = x_ref[0]
    mask_row = mask_ref[0]
    mask_col = maskc_ref[0]
    h = _ln_in(x, ln1g[...], ln1b[...])
    qk_all = jnp.dot(h, wqk[...], preferred_element_type=jnp.float32)
    v_all = jnp.dot(h, wv[...], preferred_element_type=jnp.float32)
    dh = D // heads
    NR = _NH * 32
    scale = 1.0 / jnp.sqrt(float(dh))
    lr = jax.lax.broadcasted_iota(jnp.int32, (D, D), 0) // dh
    lc = jax.lax.broadcasted_iota(jnp.int32, (D, D), 1) // dh
    band_ones = (lr == lc).astype(jnp.float32)
    norm2 = jnp.dot(qk_all * qk_all, band_ones,
                    preferred_element_type=jnp.float32)
    kn_all = qk_all / (jnp.sqrt(norm2) + 1e-6) * scale
    projT = jax.lax.dot_general(
        wrot[...], qk_all, (((0,), (1,)), ((), ())),
        preferred_element_type=jnp.float32)
    rowT = jax.lax.broadcasted_iota(jnp.int32, (L, L), 0)
    colT = jax.lax.broadcasted_iota(jnp.int32, (L, L), 1)
    ndiag_causalT = (colT > rowT) & (mask_col > 0.0)
    fillT = jnp.where(colT == rowT, -1e5, -1e9)
    ids32T = jax.lax.broadcasted_iota(jnp.int32, (32, L), 0)
    ids64T = jax.lax.broadcasted_iota(jnp.int32, (64, L), 0)
    lane_head = jax.lax.broadcasted_iota(jnp.int32, (1, D), 1) // dh
    kn_pieces = []
    v_pieces = []
    for hd in range(heads):
        hm = (lane_head == hd).astype(jnp.float32)
        kn_pieces.append(kn_all * hm)
        v_pieces.append(v_all * hm)
    knstack = jnp.concatenate(kn_pieces, axis=0)
    vstack = jnp.concatenate(v_pieces, axis=0)
    scoresT_full = jax.lax.dot_general(
        knstack, qk_all, (((1,), (1,)), ((), ())),
        preferred_element_type=jnp.float32)
    attn_pieces = []
    for hd in range(heads):
        ot_pieces = []
        for r in range(_NH):
            off = hd * NR + r * 32
            prT = projT[off:off + 32, :]
            mxpT = jnp.max(prT, axis=0, keepdims=True)
            mxnT = jnp.max(-prT, axis=0, keepdims=True)
            idpT = jnp.min(jnp.where(prT >= mxpT, ids32T, 32),
                           axis=0, keepdims=True)
            idnT = jnp.min(jnp.where(-prT >= mxnT, ids32T, 32),
                           axis=0, keepdims=True)
            bkT = jnp.where(mxpT >= mxnT, idpT, idnT + 32)
            ot_pieces.append((ids64T == bkT).astype(jnp.bfloat16))
        ot = jnp.concatenate(ot_pieces, axis=0)
        count = jax.lax.dot_general(
            ot, ot, (((0,), (0,)), ((), ())),
            preferred_element_type=jnp.float32)
        allowed = (count > 0.5) & ndiag_causalT
        sT = jnp.where(allowed, scoresT_full[hd * L:(hd + 1) * L, :], fillT)
        mxs = jnp.max(sT, axis=0, keepdims=True)
        e = jnp.exp(sT - mxs)
        attn_pieces.append(e / jnp.sum(e, axis=0, keepdims=True))
    attn_fullT = jnp.concatenate(attn_pieces, axis=0)
    o = jax.lax.dot_general(
        attn_fullT, vstack, (((0,), (0,)), ((), ())),
        preferred_element_type=jnp.float32)
    o = jnp.dot(o, wo[...], preferred_element_type=jnp.float32)
    x = x + o * mask_col
    h2 = _ln_in(x, ln2g[...], ln2b[...])
    ff = jax.nn.gelu(jnp.dot(h2, ff1w[...],
                             preferred_element_type=jnp.float32) + ff1b[...])
    ff = jnp.dot(ff, ff2w[...], preferred_element_type=jnp.float32) + ff2b[...]
    out_ref[0] = x + ff


def _lsh_block(x, mask, p, heads):
    Bb, L, D = x.shape
    dh = D // heads
    rot = jnp.transpose(p['rot'], (1, 0, 2)).reshape(dh, _NH * 32)
    rot = jnp.kron(jnp.eye(heads, dtype=jnp.float32), rot)
    mask3 = mask.reshape(Bb, 1, L)
    maskc3 = mask.reshape(Bb, L, 1)
    row1 = lambda a: a.reshape(1, -1)
    args = (x, mask3, maskc3,
            row1(p['ln1_g']), row1(p['ln1_b']),
            p['wqk'], p['wv'], p['wo'], rot,
            row1(p['ln2_g']), row1(p['ln2_b']),
            p['ff1_w'], row1(p['ff1_b']), p['ff2_w'], row1(p['ff2_b']))
    batch_spec_x = pl.BlockSpec((1, L, D), lambda b: (b, 0, 0))
    batch_spec_m = pl.BlockSpec((1, 1, L), lambda b: (b, 0, 0))
    batch_spec_mc = pl.BlockSpec((1, L, 1), lambda b: (b, 0, 0))
    w_spec = lambda a: pl.BlockSpec(a.shape, lambda b, _n=a.ndim: (0,) * _n)
    in_specs = ([batch_spec_x, batch_spec_m, batch_spec_mc]
                + [w_spec(a) for a in args[3:]])
    return pl.pallas_call(
        functools.partial(_lsh_body, heads=heads, L=L, D=D),
        grid=(Bb,),
        in_specs=in_specs,
        out_specs=batch_spec_x,
        out_shape=jax.ShapeDtypeStruct((Bb, L, D), jnp.float32),
        interpret=_INTERPRET,
    )(*args)


def _embed_body(cc_ref, cw_ref, char_w, char_b, pw_a, pw_b, proj_b,
                hw0, hb0, gw0, gb0, hw1, hb1, gw1, gb1, out_ref, *, L):
    cw = cw_ref[0]
    ch = None
    for k in range(_LCH):
        cck = cc_ref[0, :, k, :]
        t = jax.nn.relu(jnp.dot(cck, char_w[...],
                                preferred_element_type=jnp.float32)
                        + char_b[...])
        ch = t if ch is None else jnp.maximum(ch, t)
    x = (jnp.dot(cw, pw_a[...], preferred_element_type=jnp.float32)
         + jnp.dot(ch, pw_b[...], preferred_element_type=jnp.float32)
         + proj_b[...])
    for hw, hb, gw, gb in ((hw0, hb0, gw0, gb0), (hw1, hb1, gw1, gb1)):
        g = jax.nn.sigmoid(jnp.dot(x, gw[...],
                                   preferred_element_type=jnp.float32)
                           + gb[...])
        h = jax.nn.relu(jnp.dot(x, hw[...],
                                preferred_element_type=jnp.float32)
                        + hb[...])
        x = g * h + (1.0 - g) * x
    out_ref[0] = x


def _embed(cc, cw, p):
    Bb, L = cw.shape[0], cw.shape[1]
    DW = cw.shape[2]
    DC = cc.shape[3]
    row1 = lambda a: a.reshape(1, -1)
    pw = p['proj_w']
    args = (cc, cw, p['char_w'], row1(p['char_b']),
            pw[:DW], pw[DW:], row1(p['proj_b']),
            p['hw0'], row1(p['hb0']), p['gw0'], row1(p['gb0']),
            p['hw1'], row1(p['hb1']), p['gw1'], row1(p['gb1']))
    cc_spec = pl.BlockSpec((1, L, _LCH, DC), lambda b: (b, 0, 0, 0))
    cw_spec = pl.BlockSpec((1, L, DW), lambda b: (b, 0, 0))
    w_spec = lambda a: pl.BlockSpec(a.shape, lambda b, _n=a.ndim: (0,) * _n)
    in_specs = [cc_spec, cw_spec] + [w_spec(a) for a in args[2:]]
    return pl.pallas_call(
        functools.partial(_embed_body, L=L),
        grid=(Bb,),
        in_specs=in_specs,
        out_specs=pl.BlockSpec((1, L, _HID), lambda b: (b, 0, 0)),
        out_shape=jax.ShapeDtypeStruct((Bb, L, _HID), jnp.float32),
        interpret=_INTERPRET,
    )(*args)


def _bidaf_body(ce_ref, qe_ref, mc_ref, mq_ref, w4c, w4q, w4m, cqb,
                rzw, rzb, out_ref):
    ce = ce_ref[0]
    qe = qe_ref[0]
    mc_col = mc_ref[0]
    mq = mq_ref[0]
    sc = jnp.dot(ce, w4c[...], preferred_element_type=jnp.float32)
    sq = jax.lax.dot_general(
        w4q[...], qe, (((0,), (1,)), ((), ())),
        preferred_element_type=jnp.float32)
    sm = jax.lax.dot_general(
        ce * jnp.reshape(w4m[...], (1, -1)), qe,
        (((1,), (1,)), ((), ())), preferred_element_type=jnp.float32)
    S = sc + sq + sm + cqb[0, 0]
    s1 = jnp.where(mq > 0.0, S, -1e30)
    m1 = jnp.max(s1, axis=1, keepdims=True)
    e1 = jnp.exp(s1 - m1)
    S1 = e1 / jnp.sum(e1, axis=1, keepdims=True)
    s2 = jnp.where(mc_col > 0.0, S, -1e30)
    m2 = jnp.max(s2, axis=0, keepdims=True)
    e2 = jnp.exp(s2 - m2)
    S2 = e2 / jnp.sum(e2, axis=0, keepdims=True)
    A = jnp.dot(S1, qe, preferred_element_type=jnp.float32)
    t = jax.lax.dot_general(
        S2, ce, (((0,), (0,)), ((), ())), preferred_element_type=jnp.float32)
    Bm = jnp.dot(S1, t, preferred_element_type=jnp.float32)
    D = ce.shape[1]
    rz = rzw[...]
    r0 = rz[0 * D:1 * D]
    r1 = rz[1 * D:2 * D]
    r2 = rz[2 * D:3 * D]
    r3 = rz[3 * D:4 * D]
    M0 = (jnp.dot(ce, r0, preferred_element_type=jnp.float32)
          + jnp.dot(A, r1, preferred_element_type=jnp.float32)
          + jnp.dot(ce * A, r2, preferred_element_type=jnp.float32)
          + jnp.dot(ce * Bm, r3, preferred_element_type=jnp.float32)
          + rzb[...])
    out_ref[0] = M0


def _bidaf(ce, qe, mask_c, mask_q, cq, rz_w, rz_b):
    Bb, LC, D = ce.shape
    LQ = qe.shape[1]
    args = (ce, qe, mask_c.reshape(Bb, LC, 1), mask_q.reshape(Bb, 1, LQ),
            cq['w4c'].reshape(-1, 1), cq['w4q'].reshape(-1, 1),
            cq['w4m'].reshape(1, -1), cq['cq_b'].reshape(1, 1),
            rz_w, rz_b.reshape(1, -1))
    specs = [pl.BlockSpec((1, LC, D), lambda b: (b, 0, 0)),
             pl.BlockSpec((1, LQ, D), lambda b: (b, 0, 0)),
             pl.BlockSpec((1, LC, 1), lambda b: (b, 0, 0)),
             pl.BlockSpec((1, 1, LQ), lambda b: (b, 0, 0))]
    w_spec = lambda a: pl.BlockSpec(a.shape, lambda b, _n=a.ndim: (0,) * _n)
    specs += [w_spec(a) for a in args[4:]]
    return pl.pallas_call(
        _bidaf_body,
        grid=(Bb,),
        in_specs=specs,
        out_specs=pl.BlockSpec((1, LC, D), lambda b: (b, 0, 0)),
        out_shape=jax.ShapeDtypeStruct((Bb, LC, D), jnp.float32),
        interpret=_INTERPRET,
    )(*args)


def _logits_body(m1_ref, m2_ref, m3_ref, mc_ref, p1a, p1b, p2a, p2b,
                 out1_ref, out2_ref):
    m1 = m1_ref[0]
    m2 = m2_ref[0]
    m3 = m3_ref[0]
    mc = mc_ref[0]
    def head(ma, mb, wa, wb):
        la = jax.lax.dot_general(wa[...], ma, (((0,), (1,)), ((), ())),
                                 preferred_element_type=jnp.float32)
        lb = jax.lax.dot_general(wb[...], mb, (((0,), (1,)), ((), ())),
                                 preferred_element_type=jnp.float32)
        l = jnp.where(mc > 0.0, la + lb, -1e30)
        mx = jnp.max(l, axis=1, keepdims=True)
        return l - mx - jnp.log(jnp.sum(jnp.exp(l - mx), axis=1,
                                        keepdims=True))
    out1_ref[0] = head(m1, m2, p1a, p1b)
    out2_ref[0] = head(m1, m3, p2a, p2b)


def _logits(m1, m2, m3, mask_c, p1_w, p2_w):
    Bb, LC, D = m1.shape
    args = (m1, m2, m3, mask_c.reshape(Bb, 1, LC),
            p1_w[:D].reshape(-1, 1), p1_w[D:].reshape(-1, 1),
            p2_w[:D].reshape(-1, 1), p2_w[D:].reshape(-1, 1))
    x_spec = pl.BlockSpec((1, LC, D), lambda b: (b, 0, 0))
    m_spec = pl.BlockSpec((1, 1, LC), lambda b: (b, 0, 0))
    w_spec = lambda a: pl.BlockSpec(a.shape, lambda b, _n=a.ndim: (0,) * _n)
    specs = [x_spec, x_spec, x_spec, m_spec] + [w_spec(a) for a in args[4:]]
    o_spec = pl.BlockSpec((1, 1, LC), lambda b: (b, 0, 0))
    o1, o2 = pl.pallas_call(
        _logits_body,
        grid=(Bb,),
        in_specs=specs,
        out_specs=[o_spec, o_spec],
        out_shape=[jax.ShapeDtypeStruct((Bb, 1, LC), jnp.float32),
                   jax.ShapeDtypeStruct((Bb, 1, LC), jnp.float32)],
        interpret=_INTERPRET,
    )(*args)
    return o1.reshape(Bb, LC), o2.reshape(Bb, LC)


def kernel(Cwid, Qwid, Ccid, Qcid, CQid, params):
    p = params
    maskC = (Cwid != 0).astype(jnp.float32)
    maskQ = (Qwid != 0).astype(jnp.float32)
    Cw = jnp.take(p['word_table'], Cwid, axis=0)
    Qw = jnp.take(p['word_table'], Qwid, axis=0)
    Cc = jnp.take(p['char_table'], Ccid, axis=0)
    Qc = jnp.take(p['char_table'], Qcid, axis=0)
    C = _embed(Cc, Cw, p['emb'])
    Q = _embed(Qc, Qw, p['emb'])
    Ce = _lsh_block(C, maskC, p['enc'], 4)
    Qe = _lsh_block(Q, maskQ, p['enc'], 4)
    M0 = _bidaf(Ce, Qe, maskC, maskQ, p['cq'], p['rz_w'], p['rz_b'])
    for blk in p['blocks']:
        M0 = _lsh_block(M0, maskC, blk, 8)
    M1 = M0
    for blk in p['blocks']:
        M0 = _lsh_block(M0, maskC, blk, 8)
    M2 = M0
    for blk in p['blocks']:
        M0 = _lsh_block(M0, maskC, blk, 8)
    M3 = M0
    return _logits(M1, M2, M3, maskC, p['out']['p1_w'], p['out']['p2_w'])


# speedup vs baseline: 4.9938x; 1.0131x over previous
import functools

import jax
import jax.numpy as jnp
from jax.experimental import pallas as pl

_B, _LC, _LQ, _LCH = 8, 512, 64, 16
_NH = 16
_HID = 128

_INTERPRET = False


def _f32(x):
    return x.astype(jnp.float32)


def _ln_in(x, g, b):
    m = jnp.mean(x, -1, keepdims=True)
    v = jnp.var(x, -1, keepdims=True)
    return (x - m) / jnp.sqrt(v + 1e-5) * g + b


def _lsh_body(x_ref, mask_ref, maskc_ref, ln1g, ln1b, wqk, wv, wo, wrot,
              ln2g, ln2b, ff1w, ff1b, ff2w, ff2b, out_ref, *, heads, L, D):
    x = x_ref[0]
    mask_row = mask_ref[0]
    mask_col = maskc_ref[0]
    h = _ln_in(x, ln1g[...], ln1b[...])
    qk_all = jnp.dot(h, wqk[...], preferred_element_type=jnp.float32)
    v_all = jnp.dot(h, wv[...], preferred_element_type=jnp.float32)
    dh = D // heads
    NR = _NH * 32
    scale = 1.0 / jnp.sqrt(float(dh))
    lr = jax.lax.broadcasted_iota(jnp.int32, (D, D), 0) // dh
    lc = jax.lax.broadcasted_iota(jnp.int32, (D, D), 1) // dh
    band_ones = (lr == lc).astype(jnp.float32)
    norm2 = jnp.dot(qk_all * qk_all, band_ones,
                    preferred_element_type=jnp.float32)
    kn_all = qk_all / (jnp.sqrt(norm2) + 1e-6) * scale
    projT = jax.lax.dot_general(
        wrot[...], qk_all, (((0,), (1,)), ((), ())),
        preferred_element_type=jnp.float32)
    rowT = jax.lax.broadcasted_iota(jnp.int32, (L, L), 0)
    colT = jax.lax.broadcasted_iota(jnp.int32, (L, L), 1)
    ndiag_causalT = (colT > rowT) & (mask_col > 0.0)
    fillT = jnp.where(colT == rowT, -1e5, -1e9)
    ids32T = jax.lax.broadcasted_iota(jnp.int32, (32, L), 0)
    ids64T = jax.lax.broadcasted_iota(jnp.int32, (64, L), 0)
    lane_head = jax.lax.broadcasted_iota(jnp.int32, (1, D), 1) // dh
    kn_pieces = []
    v_pieces = []
    for hd in range(heads):
        hm = (lane_head == hd).astype(jnp.float32)
        kn_pieces.append((kn_all * hm).astype(jnp.bfloat16))
        v_pieces.append((v_all * hm).astype(jnp.bfloat16))
    knstack = jnp.concatenate(kn_pieces, axis=0)
    vstack = jnp.concatenate(v_pieces, axis=0)
    qk_b = qk_all.astype(jnp.bfloat16)
    scoresT_full = jax.lax.dot_general(
        knstack, qk_b, (((1,), (1,)), ((), ())),
        preferred_element_type=jnp.float32)
    attn_pieces = []
    for hd in range(heads):
        ot_pieces = []
        for r in range(_NH):
            off = hd * NR + r * 32
            prT = projT[off:off + 32, :]
            mxpT = jnp.max(prT, axis=0, keepdims=True)
            mxnT = jnp.max(-prT, axis=0, keepdims=True)
            idpT = jnp.min(jnp.where(prT >= mxpT, ids32T, 32),
                           axis=0, keepdims=True)
            idnT = jnp.min(jnp.where(-prT >= mxnT, ids32T, 32),
                           axis=0, keepdims=True)
            bkT = jnp.where(mxpT >= mxnT, idpT, idnT + 32)
            ot_pieces.append((ids64T == bkT).astype(jnp.bfloat16))
        ot = jnp.concatenate(ot_pieces, axis=0)
        count = jax.lax.dot_general(
            ot, ot, (((0,), (0,)), ((), ())),
            preferred_element_type=jnp.float32)
        allowed = (count > 0.5) & ndiag_causalT
        sT = jnp.where(allowed, scoresT_full[hd * L:(hd + 1) * L, :], fillT)
        mxs = jnp.max(sT, axis=0, keepdims=True)
        e = jnp.exp(sT - mxs)
        attn_pieces.append(
            (e / jnp.sum(e, axis=0, keepdims=True)).astype(jnp.bfloat16))
    attn_fullT = jnp.concatenate(attn_pieces, axis=0)
    o = jax.lax.dot_general(
        attn_fullT, vstack, (((0,), (0,)), ((), ())),
        preferred_element_type=jnp.float32)
    o = jnp.dot(o, wo[...], preferred_element_type=jnp.float32)
    x = x + o * mask_col
    h2 = _ln_in(x, ln2g[...], ln2b[...])
    ff = jax.nn.gelu(jnp.dot(h2, ff1w[...],
                             preferred_element_type=jnp.float32) + ff1b[...])
    ff = jnp.dot(ff, ff2w[...], preferred_element_type=jnp.float32) + ff2b[...]
    out_ref[0] = x + ff


def _lsh_block(x, mask, p, heads):
    Bb, L, D = x.shape
    dh = D // heads
    rot = jnp.transpose(p['rot'], (1, 0, 2)).reshape(dh, _NH * 32)
    rot = jnp.kron(jnp.eye(heads, dtype=jnp.float32), rot)
    mask3 = mask.reshape(Bb, 1, L)
    maskc3 = mask.reshape(Bb, L, 1)
    row1 = lambda a: a.reshape(1, -1)
    args = (x, mask3, maskc3,
            row1(p['ln1_g']), row1(p['ln1_b']),
            p['wqk'], p['wv'], p['wo'], rot,
            row1(p['ln2_g']), row1(p['ln2_b']),
            p['ff1_w'], row1(p['ff1_b']), p['ff2_w'], row1(p['ff2_b']))
    batch_spec_x = pl.BlockSpec((1, L, D), lambda b: (b, 0, 0))
    batch_spec_m = pl.BlockSpec((1, 1, L), lambda b: (b, 0, 0))
    batch_spec_mc = pl.BlockSpec((1, L, 1), lambda b: (b, 0, 0))
    w_spec = lambda a: pl.BlockSpec(a.shape, lambda b, _n=a.ndim: (0,) * _n)
    in_specs = ([batch_spec_x, batch_spec_m, batch_spec_mc]
                + [w_spec(a) for a in args[3:]])
    return pl.pallas_call(
        functools.partial(_lsh_body, heads=heads, L=L, D=D),
        grid=(Bb,),
        in_specs=in_specs,
        out_specs=batch_spec_x,
        out_shape=jax.ShapeDtypeStruct((Bb, L, D), jnp.float32),
        interpret=_INTERPRET,
    )(*args)


def _embed_body(cc_ref, cw_ref, char_w, char_b, pw_a, pw_b, proj_b,
                hw0, hb0, gw0, gb0, hw1, hb1, gw1, gb1, out_ref, *, L):
    cw = cw_ref[0]
    ch = None
    for k in range(_LCH):
        cck = cc_ref[0, :, k, :]
        t = jax.nn.relu(jnp.dot(cck, char_w[...],
                                preferred_element_type=jnp.float32)
                        + char_b[...])
        ch = t if ch is None else jnp.maximum(ch, t)
    x = (jnp.dot(cw, pw_a[...], preferred_element_type=jnp.float32)
         + jnp.dot(ch, pw_b[...], preferred_element_type=jnp.float32)
         + proj_b[...])
    for hw, hb, gw, gb in ((hw0, hb0, gw0, gb0), (hw1, hb1, gw1, gb1)):
        g = jax.nn.sigmoid(jnp.dot(x, gw[...],
                                   preferred_element_type=jnp.float32)
                           + gb[...])
        h = jax.nn.relu(jnp.dot(x, hw[...],
                                preferred_element_type=jnp.float32)
                        + hb[...])
        x = g * h + (1.0 - g) * x
    out_ref[0] = x


def _embed(cc, cw, p):
    Bb, L = cw.shape[0], cw.shape[1]
    DW = cw.shape[2]
    DC = cc.shape[3]
    row1 = lambda a: a.reshape(1, -1)
    pw = p['proj_w']
    args = (cc, cw, p['char_w'], row1(p['char_b']),
            pw[:DW], pw[DW:], row1(p['proj_b']),
            p['hw0'], row1(p['hb0']), p['gw0'], row1(p['gb0']),
            p['hw1'], row1(p['hb1']), p['gw1'], row1(p['gb1']))
    cc_spec = pl.BlockSpec((1, L, _LCH, DC), lambda b: (b, 0, 0, 0))
    cw_spec = pl.BlockSpec((1, L, DW), lambda b: (b, 0, 0))
    w_spec = lambda a: pl.BlockSpec(a.shape, lambda b, _n=a.ndim: (0,) * _n)
    in_specs = [cc_spec, cw_spec] + [w_spec(a) for a in args[2:]]
    return pl.pallas_call(
        functools.partial(_embed_body, L=L),
        grid=(Bb,),
        in_specs=in_specs,
        out_specs=pl.BlockSpec((1, L, _HID), lambda b: (b, 0, 0)),
        out_shape=jax.ShapeDtypeStruct((Bb, L, _HID), jnp.float32),
        interpret=_INTERPRET,
    )(*args)


def _bidaf_body(ce_ref, qe_ref, mc_ref, mq_ref, w4c, w4q, w4m, cqb,
                rzw, rzb, out_ref):
    ce = ce_ref[0]
    qe = qe_ref[0]
    mc_col = mc_ref[0]
    mq = mq_ref[0]
    sc = jnp.dot(ce, w4c[...], preferred_element_type=jnp.float32)
    sq = jax.lax.dot_general(
        w4q[...], qe, (((0,), (1,)), ((), ())),
        preferred_element_type=jnp.float32)
    sm = jax.lax.dot_general(
        ce * jnp.reshape(w4m[...], (1, -1)), qe,
        (((1,), (1,)), ((), ())), preferred_element_type=jnp.float32)
    S = sc + sq + sm + cqb[0, 0]
    s1 = jnp.where(mq > 0.0, S, -1e30)
    m1 = jnp.max(s1, axis=1, keepdims=True)
    e1 = jnp.exp(s1 - m1)
    S1 = e1 / jnp.sum(e1, axis=1, keepdims=True)
    s2 = jnp.where(mc_col > 0.0, S, -1e30)
    m2 = jnp.max(s2, axis=0, keepdims=True)
    e2 = jnp.exp(s2 - m2)
    S2 = e2 / jnp.sum(e2, axis=0, keepdims=True)
    A = jnp.dot(S1, qe, preferred_element_type=jnp.float32)
    t = jax.lax.dot_general(
        S2, ce, (((0,), (0,)), ((), ())), preferred_element_type=jnp.float32)
    Bm = jnp.dot(S1, t, preferred_element_type=jnp.float32)
    D = ce.shape[1]
    rz = rzw[...]
    r0 = rz[0 * D:1 * D]
    r1 = rz[1 * D:2 * D]
    r2 = rz[2 * D:3 * D]
    r3 = rz[3 * D:4 * D]
    M0 = (jnp.dot(ce, r0, preferred_element_type=jnp.float32)
          + jnp.dot(A, r1, preferred_element_type=jnp.float32)
          + jnp.dot(ce * A, r2, preferred_element_type=jnp.float32)
          + jnp.dot(ce * Bm, r3, preferred_element_type=jnp.float32)
          + rzb[...])
    out_ref[0] = M0


def _bidaf(ce, qe, mask_c, mask_q, cq, rz_w, rz_b):
    Bb, LC, D = ce.shape
    LQ = qe.shape[1]
    args = (ce, qe, mask_c.reshape(Bb, LC, 1), mask_q.reshape(Bb, 1, LQ),
            cq['w4c'].reshape(-1, 1), cq['w4q'].reshape(-1, 1),
            cq['w4m'].reshape(1, -1), cq['cq_b'].reshape(1, 1),
            rz_w, rz_b.reshape(1, -1))
    specs = [pl.BlockSpec((1, LC, D), lambda b: (b, 0, 0)),
             pl.BlockSpec((1, LQ, D), lambda b: (b, 0, 0)),
             pl.BlockSpec((1, LC, 1), lambda b: (b, 0, 0)),
             pl.BlockSpec((1, 1, LQ), lambda b: (b, 0, 0))]
    w_spec = lambda a: pl.BlockSpec(a.shape, lambda b, _n=a.ndim: (0,) * _n)
    specs += [w_spec(a) for a in args[4:]]
    return pl.pallas_call(
        _bidaf_body,
        grid=(Bb,),
        in_specs=specs,
        out_specs=pl.BlockSpec((1, LC, D), lambda b: (b, 0, 0)),
        out_shape=jax.ShapeDtypeStruct((Bb, LC, D), jnp.float32),
        interpret=_INTERPRET,
    )(*args)


def _logits_body(m1_ref, m2_ref, m3_ref, mc_ref, p1a, p1b, p2a, p2b,
                 out1_ref, out2_ref):
    m1 = m1_ref[0]
    m2 = m2_ref[0]
    m3 = m3_ref[0]
    mc = mc_ref[0]
    def head(ma, mb, wa, wb):
        la = jax.lax.dot_general(wa[...], ma, (((0,), (1,)), ((), ())),
                                 preferred_element_type=jnp.float32)
        lb = jax.lax.dot_general(wb[...], mb, (((0,), (1,)), ((), ())),
                                 preferred_element_type=jnp.float32)
        l = jnp.where(mc > 0.0, la + lb, -1e30)
        mx = jnp.max(l, axis=1, keepdims=True)
        return l - mx - jnp.log(jnp.sum(jnp.exp(l - mx), axis=1,
                                        keepdims=True))
    out1_ref[0] = head(m1, m2, p1a, p1b)
    out2_ref[0] = head(m1, m3, p2a, p2b)


def _logits(m1, m2, m3, mask_c, p1_w, p2_w):
    Bb, LC, D = m1.shape
    args = (m1, m2, m3, mask_c.reshape(Bb, 1, LC),
            p1_w[:D].reshape(-1, 1), p1_w[D:].reshape(-1, 1),
            p2_w[:D].reshape(-1, 1), p2_w[D:].reshape(-1, 1))
    x_spec = pl.BlockSpec((1, LC, D), lambda b: (b, 0, 0))
    m_spec = pl.BlockSpec((1, 1, LC), lambda b: (b, 0, 0))
    w_spec = lambda a: pl.BlockSpec(a.shape, lambda b, _n=a.ndim: (0,) * _n)
    specs = [x_spec, x_spec, x_spec, m_spec] + [w_spec(a) for a in args[4:]]
    o_spec = pl.BlockSpec((1, 1, LC), lambda b: (b, 0, 0))
    o1, o2 = pl.pallas_call(
        _logits_body,
        grid=(Bb,),
        in_specs=specs,
        out_specs=[o_spec, o_spec],
        out_shape=[jax.ShapeDtypeStruct((Bb, 1, LC), jnp.float32),
                   jax.ShapeDtypeStruct((Bb, 1, LC), jnp.float32)],
        interpret=_INTERPRET,
    )(*args)
    return o1.reshape(Bb, LC), o2.reshape(Bb, LC)


def kernel(Cwid, Qwid, Ccid, Qcid, CQid, params):
    p = params
    maskC = (Cwid != 0).astype(jnp.float32)
    maskQ = (Qwid != 0).astype(jnp.float32)
    Cw = jnp.take(p['word_table'], Cwid, axis=0)
    Qw = jnp.take(p['word_table'], Qwid, axis=0)
    Cc = jnp.take(p['char_table'], Ccid, axis=0)
    Qc = jnp.take(p['char_table'], Qcid, axis=0)
    C = _embed(Cc, Cw, p['emb'])
    Q = _embed(Qc, Qw, p['emb'])
    Ce = _lsh_block(C, maskC, p['enc'], 4)
    Qe = _lsh_block(Q, maskQ, p['enc'], 4)
    M0 = _bidaf(Ce, Qe, maskC, maskQ, p['cq'], p['rz_w'], p['rz_b'])
    for blk in p['blocks']:
        M0 = _lsh_block(M0, maskC, blk, 8)
    M1 = M0
    for blk in p['blocks']:
        M0 = _lsh_block(M0, maskC, blk, 8)
    M2 = M0
    for blk in p['blocks']:
        M0 = _lsh_block(M0, maskC, blk, 8)
    M3 = M0
    return _logits(M1, M2, M3, maskC, p['out']['p1_w'], p['out']['p2_w'])
